# baseline (speedup 1.0000x reference)
import functools

import jax
import jax.numpy as jnp
from jax import lax
from jax.experimental import pallas as pl
from jax.experimental.pallas import tpu as pltpu
from jax.experimental.pallas import tpu_sc as plsc

N = 10000
E = 320000
NC = 2
NS = 16
L = 16
NW = NC * NS
EPS = E // NS
N_PAD = 10240
NRED = N_PAD // NS

HFETCH = 20096
HF0 = 9984
HF1 = HFETCH - HF0
SC0 = 618
SCALL = EPS // L
GLEN = 10112
GCH = GLEN // L

ZU = 8
SU = 16
GU = 16

_mesh = plsc.VectorSubcoreMesh(core_axis_name="c", subcore_axis_name="s")
_cparams = pltpu.CompilerParams(needs_layout_passes=False)


@functools.partial(
    pl.kernel,
    mesh=_mesh,
    compiler_params=_cparams,
    out_type=jax.ShapeDtypeStruct((1, E), jnp.float32),
    scratch_types=[
        pltpu.VMEM((2, HFETCH), jnp.int32),
        pltpu.VMEM((2, GLEN), jnp.int32),
        pltpu.VMEM((N_PAD,), jnp.float32),
        pltpu.VMEM_SHARED((NS, N_PAD), jnp.float32),
        pltpu.VMEM_SHARED((N_PAD,), jnp.float32),
        pltpu.VMEM((NS, NRED), jnp.float32),
        pltpu.VMEM((NRED,), jnp.float32),
        pltpu.VMEM((N_PAD,), jnp.float32),
        pltpu.VMEM((GLEN,), jnp.float32),
        pltpu.SemaphoreType.DMA,
        pltpu.SemaphoreType.DMA,
    ],
)
def _alpha_k(edge_hbm, out_hbm, idx_v, gidx_v, hist_v, part_sh, recip_sh,
             red_v, outred_v, recip_v, out_v, sem, gsem):
    c = lax.axis_index("c")
    s = lax.axis_index("s")
    w = s * NC + c

    col = s * EPS
    c0 = (col // 128) * 128
    off = col - c0
    cp0 = pltpu.async_copy(
        edge_hbm.at[:, pl.ds(c0, HF0)], idx_v.at[:, pl.ds(0, HF0)], sem)
    ga = ((625 * w) // 8) * 128

    @pl.when(c == 1)
    def _():
        pltpu.async_copy(edge_hbm.at[:, pl.ds(ga, GLEN)], gidx_v, gsem)

    def zero_body(i, carry):
        base = i * (L * ZU)
        for u in range(ZU):
            hist_v[pl.ds(base + u * L, L)] = jnp.zeros((L,), jnp.float32)
        return carry

    lax.fori_loop(0, N_PAD // (L * ZU), zero_body, 0)
    cp0.wait()
    cp1 = pltpu.async_copy(
        edge_hbm.at[:, pl.ds(c0 + HF0, HF1)], idx_v.at[:, pl.ds(HF0, HF1)], sem)

    ones = jnp.ones((L,), jnp.float32)

    def scat_body(i):
        idx = idx_v[1, pl.ds(off + i * L, L)]
        plsc.addupdate_scatter(hist_v, [idx], ones)

    plsc.parallel_loop(0, SC0, unroll=SU)(scat_body)
    cp1.wait()
    plsc.parallel_loop(SC0, SCALL, unroll=SU)(scat_body)

    pltpu.sync_copy(hist_v, part_sh.at[s])
    plsc.subcore_barrier()
    pltpu.sync_copy(part_sh.at[:, pl.ds(s * NRED, NRED)], red_v)

    one = jnp.ones((L,), jnp.float32)

    def red_body(i, carry):
        acc = red_v[0, pl.ds(i * L, L)]
        for r in range(1, NS):
            acc = acc + red_v[r, pl.ds(i * L, L)]
        outred_v[pl.ds(i * L, L)] = one / acc
        return carry

    lax.fori_loop(0, NRED // L, red_body, 0)
    pltpu.sync_copy(outred_v, recip_sh.at[pl.ds(s * NRED, NRED)])
    plsc.subcore_barrier()

    pltpu.sync_copy(recip_sh, recip_v)

    @pl.when(c == 0)
    def _():
        def g_body0(i):
            idx = idx_v[1, pl.ds(i * L, L)]
            out_v[pl.ds(i * L, L)] = plsc.load_gather(recip_v, [idx])

        plsc.parallel_loop(0, GCH, unroll=GU)(g_body0)

    @pl.when(c == 1)
    def _():
        pltpu.make_async_copy(
            edge_hbm.at[:, pl.ds(ga, GLEN)], gidx_v, gsem).wait()

        def g_body1(i):
            idx = gidx_v[1, pl.ds(i * L, L)]
            out_v[pl.ds(i * L, L)] = plsc.load_gather(recip_v, [idx])

        plsc.parallel_loop(0, GCH, unroll=GU)(g_body1)

    pltpu.sync_copy(out_v, out_hbm.at[0, pl.ds(ga, GLEN)])


@jax.jit
def kernel(x, edge_index, W_l, b_l, W_r, b_r, attn_w):
    return _alpha_k(edge_index.astype(jnp.int32)).reshape(E, 1)


# speedup vs baseline: 1.0215x; 1.0215x over previous
import functools

import jax
import jax.numpy as jnp
from jax import lax
from jax.experimental import pallas as pl
from jax.experimental.pallas import tpu as pltpu
from jax.experimental.pallas import tpu_sc as plsc

N = 10000
E = 320000
NC = 2
NS = 16
L = 16
NW = NC * NS
EPS = E // NS
N_PAD = 10240
NRED = N_PAD // NS

HFETCH = 20096
HF0 = 9984
HF1 = HFETCH - HF0
SC0 = 618
SCALL = EPS // L
GLEN = 10112
GCH = GLEN // L

ZU = 8
SU = 8
GU = 8

_mesh = plsc.VectorSubcoreMesh(core_axis_name="c", subcore_axis_name="s")
_cparams = pltpu.CompilerParams(needs_layout_passes=False)


@functools.partial(
    pl.kernel,
    mesh=_mesh,
    compiler_params=_cparams,
    out_type=jax.ShapeDtypeStruct((1, E), jnp.float32),
    scratch_types=[
        pltpu.VMEM((2, HFETCH), jnp.int32),
        pltpu.VMEM((2, GLEN), jnp.int32),
        pltpu.VMEM((N_PAD,), jnp.float32),
        pltpu.VMEM_SHARED((NS, N_PAD), jnp.float32),
        pltpu.VMEM_SHARED((N_PAD,), jnp.float32),
        pltpu.VMEM((NS, NRED), jnp.float32),
        pltpu.VMEM((NRED,), jnp.float32),
        pltpu.VMEM((N_PAD,), jnp.float32),
        pltpu.VMEM((GLEN,), jnp.float32),
        pltpu.SemaphoreType.DMA,
        pltpu.SemaphoreType.DMA,
    ],
)
def _alpha_k(edge_hbm, out_hbm, idx_v, gidx_v, hist_v, part_sh, recip_sh,
             red_v, outred_v, recip_v, out_v, sem, gsem):
    c = lax.axis_index("c")
    s = lax.axis_index("s")
    w = s * NC + c

    col = s * EPS
    c0 = (col // 128) * 128
    off = col - c0
    cp0 = pltpu.async_copy(
        edge_hbm.at[:, pl.ds(c0, HF0)], idx_v.at[:, pl.ds(0, HF0)], sem)
    ga = ((625 * w) // 8) * 128

    @pl.when(c == 1)
    def _():
        pltpu.async_copy(edge_hbm.at[:, pl.ds(ga, GLEN)], gidx_v, gsem)

    def zero_body(i, carry):
        base = i * (L * ZU)
        for u in range(ZU):
            hist_v[pl.ds(base + u * L, L)] = jnp.zeros((L,), jnp.float32)
        return carry

    lax.fori_loop(0, N_PAD // (L * ZU), zero_body, 0)
    cp0.wait()
    cp1 = pltpu.async_copy(
        edge_hbm.at[:, pl.ds(c0 + HF0, HF1)], idx_v.at[:, pl.ds(HF0, HF1)], sem)

    ones = jnp.ones((L,), jnp.float32)

    def scat_body(i):
        idx = idx_v[1, pl.ds(off + i * L, L)]
        plsc.addupdate_scatter(hist_v, [idx], ones)

    plsc.parallel_loop(0, SC0, unroll=SU)(scat_body)
    cp1.wait()
    plsc.parallel_loop(SC0, SCALL, unroll=SU)(scat_body)

    pltpu.sync_copy(hist_v, part_sh.at[s])
    plsc.subcore_barrier()
    pltpu.sync_copy(part_sh.at[:, pl.ds(s * NRED, NRED)], red_v)

    one = jnp.ones((L,), jnp.float32)

    def red_body(i, carry):
        acc = red_v[0, pl.ds(i * L, L)]
        for r in range(1, NS):
            acc = acc + red_v[r, pl.ds(i * L, L)]
        outred_v[pl.ds(i * L, L)] = one / acc
        return carry

    lax.fori_loop(0, NRED // L, red_body, 0)
    pltpu.sync_copy(outred_v, recip_sh.at[pl.ds(s * NRED, NRED)])
    plsc.subcore_barrier()

    pltpu.sync_copy(recip_sh, recip_v)

    @pl.when(c == 0)
    def _():
        def g_body0(i):
            idx = idx_v[1, pl.ds(i * L, L)]
            out_v[pl.ds(i * L, L)] = plsc.load_gather(recip_v, [idx])

        plsc.parallel_loop(0, GCH, unroll=GU)(g_body0)

    @pl.when(c == 1)
    def _():
        pltpu.make_async_copy(
            edge_hbm.at[:, pl.ds(ga, GLEN)], gidx_v, gsem).wait()

        def g_body1(i):
            idx = gidx_v[1, pl.ds(i * L, L)]
            out_v[pl.ds(i * L, L)] = plsc.load_gather(recip_v, [idx])

        plsc.parallel_loop(0, GCH, unroll=GU)(g_body1)

    pltpu.sync_copy(out_v, out_hbm.at[0, pl.ds(ga, GLEN)])


@jax.jit
def kernel(x, edge_index, W_l, b_l, W_r, b_r, attn_w):
    return _alpha_k(edge_index.astype(jnp.int32)).reshape(E, 1)


# speedup vs baseline: 1.0291x; 1.0075x over previous
import functools

import jax
import jax.numpy as jnp
from jax import lax
from jax.experimental import pallas as pl
from jax.experimental.pallas import tpu as pltpu
from jax.experimental.pallas import tpu_sc as plsc

N = 10000
E = 320000
NC = 2
NS = 16
L = 16
NW = NC * NS
EPS = E // NS
N_PAD = 10240
NRED = N_PAD // NS

HFETCH = 20096
FQ = (5120, 5120, 5120, 4736)
SCB = (0, 314, 634, 954, 1250)
SCALL = EPS // L
GLEN = 10112
GCH = GLEN // L

ZU = 8
SU = 8
GU = 8

_mesh = plsc.VectorSubcoreMesh(core_axis_name="c", subcore_axis_name="s")
_cparams = pltpu.CompilerParams(needs_layout_passes=False)


@functools.partial(
    pl.kernel,
    mesh=_mesh,
    compiler_params=_cparams,
    out_type=jax.ShapeDtypeStruct((1, E), jnp.float32),
    scratch_types=[
        pltpu.VMEM((2, HFETCH), jnp.int32),
        pltpu.VMEM((2, GLEN), jnp.int32),
        pltpu.VMEM((N_PAD,), jnp.float32),
        pltpu.VMEM_SHARED((NS, N_PAD), jnp.float32),
        pltpu.VMEM_SHARED((N_PAD,), jnp.float32),
        pltpu.VMEM((NS, NRED), jnp.float32),
        pltpu.VMEM((NRED,), jnp.float32),
        pltpu.VMEM((N_PAD,), jnp.float32),
        pltpu.VMEM((GLEN,), jnp.float32),
        pltpu.SemaphoreType.DMA,
        pltpu.SemaphoreType.DMA,
        pltpu.SemaphoreType.DMA,
        pltpu.SemaphoreType.DMA,
        pltpu.SemaphoreType.DMA,
    ],
)
def _alpha_k(edge_hbm, out_hbm, idx_v, gidx_v, hist_v, part_sh, recip_sh,
             red_v, outred_v, recip_v, out_v, sem0, sem1, sem2, sem3, gsem):
    c = lax.axis_index("c")
    s = lax.axis_index("s")
    w = s * NC + c

    col = s * EPS
    c0 = (col // 128) * 128
    off = col - c0
    sems = (sem0, sem1, sem2, sem3)
    starts = (0, FQ[0], FQ[0] + FQ[1], FQ[0] + FQ[1] + FQ[2])

    def fetch(j):
        return pltpu.async_copy(
            edge_hbm.at[:, pl.ds(c0 + starts[j], FQ[j])],
            idx_v.at[:, pl.ds(starts[j], FQ[j])], sems[j])

    cps = [fetch(0), fetch(1)]
    ga = ((625 * w) // 8) * 128

    @pl.when(c == 1)
    def _():
        pltpu.async_copy(edge_hbm.at[:, pl.ds(ga, GLEN)], gidx_v, gsem)

    def zero_body(i, carry):
        base = i * (L * ZU)
        for u in range(ZU):
            hist_v[pl.ds(base + u * L, L)] = jnp.zeros((L,), jnp.float32)
        return carry

    lax.fori_loop(0, N_PAD // (L * ZU), zero_body, 0)

    ones = jnp.ones((L,), jnp.float32)

    def scat_body(i):
        idx = idx_v[1, pl.ds(off + i * L, L)]
        plsc.addupdate_scatter(hist_v, [idx], ones)

    for j in range(4):
        cps[j].wait()
        if j + 2 < 4:
            cps.append(fetch(j + 2))
        plsc.parallel_loop(SCB[j], SCB[j + 1], unroll=SU)(scat_body)

    pltpu.sync_copy(hist_v, part_sh.at[s])
    plsc.subcore_barrier()
    pltpu.sync_copy(part_sh.at[:, pl.ds(s * NRED, NRED)], red_v)

    one = jnp.ones((L,), jnp.float32)

    def red_body(i, carry):
        acc = red_v[0, pl.ds(i * L, L)]
        for r in range(1, NS):
            acc = acc + red_v[r, pl.ds(i * L, L)]
        outred_v[pl.ds(i * L, L)] = one / acc
        return carry

    lax.fori_loop(0, NRED // L, red_body, 0)
    pltpu.sync_copy(outred_v, recip_sh.at[pl.ds(s * NRED, NRED)])
    plsc.subcore_barrier()

    pltpu.sync_copy(recip_sh, recip_v)

    @pl.when(c == 0)
    def _():
        def g_body0(i):
            idx = idx_v[1, pl.ds(i * L, L)]
            out_v[pl.ds(i * L, L)] = plsc.load_gather(recip_v, [idx])

        plsc.parallel_loop(0, GCH, unroll=GU)(g_body0)

    @pl.when(c == 1)
    def _():
        pltpu.make_async_copy(
            edge_hbm.at[:, pl.ds(ga, GLEN)], gidx_v, gsem).wait()

        def g_body1(i):
            idx = gidx_v[1, pl.ds(i * L, L)]
            out_v[pl.ds(i * L, L)] = plsc.load_gather(recip_v, [idx])

        plsc.parallel_loop(0, GCH, unroll=GU)(g_body1)

    pltpu.sync_copy(out_v, out_hbm.at[0, pl.ds(ga, GLEN)])


@jax.jit
def kernel(x, edge_index, W_l, b_l, W_r, b_r, attn_w):
    return _alpha_k(edge_index.astype(jnp.int32)).reshape(E, 1)
